# initial kernel scaffold (unmeasured)
import jax
import jax.numpy as jnp
from jax import lax
from jax.experimental import pallas as pl
from jax.experimental.pallas import tpu as pltpu

NZ = 4
B, S, H, Dh, Dr = 4, 256, 32, 128, 64
D = 4096
DC = 512
DCZ = DC // NZ
M = B * S
SCALE = (Dh + Dr) ** -0.5


def _dot(a, b, dn=(((1,), (0,)), ((), ()))):
    return lax.dot_general(
        a, b, dn,
        precision=lax.Precision.DEFAULT,
        preferred_element_type=jnp.float32,
    )


def _gemm_body(x_ref, w_ref, o_ref):
    o_ref[...] = _dot(x_ref[...], w_ref[...])


def _gemm(x, w, block_n=None):
    m, k = x.shape
    _, n = w.shape
    if block_n is None:
        block_n = n
    return pl.pallas_call(
        _gemm_body,
        grid=(n // block_n,),
        in_specs=[
            pl.BlockSpec((m, k), lambda j: (0, 0)),
            pl.BlockSpec((k, block_n), lambda j: (0, j)),
        ],
        out_specs=pl.BlockSpec((m, block_n), lambda j: (0, j)),
        out_shape=jax.ShapeDtypeStruct((m, n), jnp.float32),
    )(x, w)


def _gather_body(cz_ref, wuk_ref, wuv_ref, c_out, wuk_out, wuv_out,
                 send_c, recv_c, send_k, recv_k, send_v, recv_v):
    mx = lax.axis_index("x")
    my = lax.axis_index("y")
    mz = lax.axis_index("z")
    right = (mz + 1) % NZ
    left = (mz + NZ - 1) % NZ

    barrier = pltpu.get_barrier_semaphore()
    for nbr in (left, right):
        pl.semaphore_signal(
            barrier, inc=1,
            device_id=(mx, my, nbr),
            device_id_type=pl.DeviceIdType.MESH,
        )
    pl.semaphore_wait(barrier, 2)

    c_out[mz] = cz_ref[...]
    wuk_out[pl.ds(mz * DCZ, DCZ), :] = wuk_ref[...]
    wuv_out[pl.ds(mz * DCZ, DCZ), :] = wuv_ref[...]

    for h in range(NZ - 1):
        src = (mz + NZ - h) % NZ
        rd_c = pltpu.make_async_remote_copy(
            src_ref=c_out.at[src],
            dst_ref=c_out.at[src],
            send_sem=send_c.at[h],
            recv_sem=recv_c.at[h],
            device_id=(mx, my, right),
            device_id_type=pl.DeviceIdType.MESH,
        )
        rd_k = pltpu.make_async_remote_copy(
            src_ref=wuk_out.at[pl.ds(src * DCZ, DCZ)],
            dst_ref=wuk_out.at[pl.ds(src * DCZ, DCZ)],
            send_sem=send_k.at[h],
            recv_sem=recv_k.at[h],
            device_id=(mx, my, right),
            device_id_type=pl.DeviceIdType.MESH,
        )
        rd_v = pltpu.make_async_remote_copy(
            src_ref=wuv_out.at[pl.ds(src * DCZ, DCZ)],
            dst_ref=wuv_out.at[pl.ds(src * DCZ, DCZ)],
            send_sem=send_v.at[h],
            recv_sem=recv_v.at[h],
            device_id=(mx, my, right),
            device_id_type=pl.DeviceIdType.MESH,
        )
        rd_c.start()
        rd_k.start()
        rd_v.start()
        rd_c.wait()
        rd_k.wait()
        rd_v.wait()


def _gather(cz, wuk_z, wuv_z):
    return pl.pallas_call(
        _gather_body,
        out_shape=[
            jax.ShapeDtypeStruct((NZ, M, DCZ), jnp.float32),
            jax.ShapeDtypeStruct((DC, D), jnp.float32),
            jax.ShapeDtypeStruct((DC, D), jnp.float32),
        ],
        in_specs=[pl.BlockSpec(memory_space=pltpu.VMEM)] * 3,
        out_specs=[pl.BlockSpec(memory_space=pltpu.VMEM)] * 3,
        scratch_shapes=[pltpu.SemaphoreType.DMA((NZ - 1,))] * 6,
        compiler_params=pltpu.CompilerParams(collective_id=0),
    )(cz, wuk_z, wuv_z)


def _kv_body(c_ref, w_ref, o_ref):
    acc = _dot(c_ref[0], w_ref[pl.ds(0, DCZ), :])
    for z in range(1, NZ):
        acc = acc + _dot(c_ref[z], w_ref[pl.ds(z * DCZ, DCZ), :])
    o_ref[...] = acc


def _kv(c_all, w, block_n=1024):
    return pl.pallas_call(
        _kv_body,
        grid=(D // block_n,),
        in_specs=[
            pl.BlockSpec((NZ, M, DCZ), lambda j: (0, 0, 0)),
            pl.BlockSpec((DC, block_n), lambda j: (0, j)),
        ],
        out_specs=pl.BlockSpec((M, block_n), lambda j: (0, j)),
        out_shape=jax.ShapeDtypeStruct((M, D), jnp.float32),
    )(c_all, w)


def _attn_body(q_ref, qr_ref, k_ref, kr_ref, v_ref, o_ref):
    dn_t = (((1,), (1,)), ((), ()))
    s = (_dot(q_ref[...], k_ref[...], dn_t)
         + _dot(qr_ref[...], kr_ref[...], dn_t)) * SCALE
    m_ = jnp.max(s, axis=1, keepdims=True)
    p = jnp.exp(s - m_)
    p = p / jnp.sum(p, axis=1, keepdims=True)
    o_ref[...] = _dot(p, v_ref[...])


def _attention(Q, Qr, K, Kr, V):
    return pl.pallas_call(
        _attn_body,
        grid=(B, H),
        in_specs=[
            pl.BlockSpec((S, Dh), lambda b, h: (b, h)),
            pl.BlockSpec((S, Dr), lambda b, h: (b, h)),
            pl.BlockSpec((S, Dh), lambda b, h: (b, h)),
            pl.BlockSpec((S, Dr), lambda b, h: (b, 0)),
            pl.BlockSpec((S, Dh), lambda b, h: (b, h)),
        ],
        out_specs=pl.BlockSpec((S, Dh), lambda b, h: (b, h)),
        out_shape=jax.ShapeDtypeStruct((M, D), jnp.float32),
    )(Q, Qr, K, Kr, V)


def kernel(x, Wdkv, Wuk, Wuv, Wq, Wqr, Wkr, Wo):
    x2 = x.reshape(M, D)
    cz = _gemm(x2, Wdkv)
    c_all, wuk_f, wuv_f = _gather(cz, Wuk, Wuv)
    Q = _gemm(x2, Wq, 512)
    Qr = _gemm(x2, Wqr, 512)
    Kr = _gemm(x2, Wkr)
    K = _kv(c_all, wuk_f)
    V = _kv(c_all, wuv_f)
    O = _attention(Q, Qr, K, Kr, V)
    out = _gemm(O, Wo, 512)
    return out.reshape(B, S, D)


# baseline (device time: 419651 ns/iter reference)
import jax
import jax.numpy as jnp
from jax import lax
from jax.experimental import pallas as pl
from jax.experimental.pallas import tpu as pltpu

NZ = 4
B, S, H, Dh, Dr = 4, 256, 32, 128, 64
D = 4096
DC = 512
DCZ = DC // NZ
M = B * S
SCALE = (Dh + Dr) ** -0.5


def _dot(a, b, dn=(((1,), (0,)), ((), ()))):
    return lax.dot_general(
        a, b, dn,
        precision=lax.Precision.DEFAULT,
        preferred_element_type=jnp.float32,
    )


def _gemm_body(x_ref, w_ref, o_ref):
    o_ref[...] = _dot(x_ref[...], w_ref[...])


def _gemm(x, w, block_n=None):
    m, k = x.shape
    _, n = w.shape
    if block_n is None:
        block_n = n
    return pl.pallas_call(
        _gemm_body,
        grid=(n // block_n,),
        in_specs=[
            pl.BlockSpec((m, k), lambda j: (0, 0)),
            pl.BlockSpec((k, block_n), lambda j: (0, j)),
        ],
        out_specs=pl.BlockSpec((m, block_n), lambda j: (0, j)),
        out_shape=jax.ShapeDtypeStruct((m, n), jnp.float32),
    )(x, w)


def _gather_body(cz_ref, wuk_ref, wuv_ref, c_out, wuk_out, wuv_out,
                 send_c, recv_c, send_k, recv_k, send_v, recv_v):
    mx = lax.axis_index("x")
    my = lax.axis_index("y")
    mz = lax.axis_index("z")
    right = (mz + 1) % NZ
    left = (mz + NZ - 1) % NZ

    barrier = pltpu.get_barrier_semaphore()
    for nbr in (left, right):
        pl.semaphore_signal(
            barrier, inc=1,
            device_id=(mx, my, nbr),
            device_id_type=pl.DeviceIdType.MESH,
        )
    pl.semaphore_wait(barrier, 2)

    c_out[mz] = cz_ref[...]
    wuk_out[pl.ds(mz * DCZ, DCZ), :] = wuk_ref[...]
    wuv_out[pl.ds(mz * DCZ, DCZ), :] = wuv_ref[...]

    for h in range(NZ - 1):
        src = (mz + NZ - h) % NZ
        rd_c = pltpu.make_async_remote_copy(
            src_ref=c_out.at[src],
            dst_ref=c_out.at[src],
            send_sem=send_c.at[h],
            recv_sem=recv_c.at[h],
            device_id=(mx, my, right),
            device_id_type=pl.DeviceIdType.MESH,
        )
        rd_k = pltpu.make_async_remote_copy(
            src_ref=wuk_out.at[pl.ds(src * DCZ, DCZ)],
            dst_ref=wuk_out.at[pl.ds(src * DCZ, DCZ)],
            send_sem=send_k.at[h],
            recv_sem=recv_k.at[h],
            device_id=(mx, my, right),
            device_id_type=pl.DeviceIdType.MESH,
        )
        rd_v = pltpu.make_async_remote_copy(
            src_ref=wuv_out.at[pl.ds(src * DCZ, DCZ)],
            dst_ref=wuv_out.at[pl.ds(src * DCZ, DCZ)],
            send_sem=send_v.at[h],
            recv_sem=recv_v.at[h],
            device_id=(mx, my, right),
            device_id_type=pl.DeviceIdType.MESH,
        )
        rd_c.start()
        rd_k.start()
        rd_v.start()
        rd_c.wait()
        rd_k.wait()
        rd_v.wait()


def _gather(cz, wuk_z, wuv_z):
    return pl.pallas_call(
        _gather_body,
        out_shape=[
            jax.ShapeDtypeStruct((NZ, M, DCZ), jnp.float32),
            jax.ShapeDtypeStruct((DC, D), jnp.float32),
            jax.ShapeDtypeStruct((DC, D), jnp.float32),
        ],
        in_specs=[pl.BlockSpec(memory_space=pltpu.VMEM)] * 3,
        out_specs=[pl.BlockSpec(memory_space=pltpu.VMEM)] * 3,
        scratch_shapes=[pltpu.SemaphoreType.DMA((NZ - 1,))] * 6,
        compiler_params=pltpu.CompilerParams(collective_id=0),
    )(cz, wuk_z, wuv_z)


def _kv_body(c_ref, w_ref, o_ref):
    acc = _dot(c_ref[0], w_ref[pl.ds(0, DCZ), :])
    for z in range(1, NZ):
        acc = acc + _dot(c_ref[z], w_ref[pl.ds(z * DCZ, DCZ), :])
    o_ref[...] = acc


def _kv(c_all, w, block_n=1024):
    return pl.pallas_call(
        _kv_body,
        grid=(D // block_n,),
        in_specs=[
            pl.BlockSpec((NZ, M, DCZ), lambda j: (0, 0, 0)),
            pl.BlockSpec((DC, block_n), lambda j: (0, j)),
        ],
        out_specs=pl.BlockSpec((M, block_n), lambda j: (0, j)),
        out_shape=jax.ShapeDtypeStruct((M, D), jnp.float32),
    )(c_all, w)


HPB = 2


def _attn_body(q_ref, qr_ref, k_ref, kr_ref, v_ref, o_ref):
    dn_t = (((1,), (1,)), ((), ()))
    kr = kr_ref[...]
    for i in range(HPB):
        q = q_ref[:, i * Dh:(i + 1) * Dh]
        qr = qr_ref[:, i * Dr:(i + 1) * Dr]
        k = k_ref[:, i * Dh:(i + 1) * Dh]
        v = v_ref[:, i * Dh:(i + 1) * Dh]
        s = (_dot(q, k, dn_t) + _dot(qr, kr, dn_t)) * SCALE
        m_ = jnp.max(s, axis=1, keepdims=True)
        p = jnp.exp(s - m_)
        p = p / jnp.sum(p, axis=1, keepdims=True)
        o_ref[:, i * Dh:(i + 1) * Dh] = _dot(p, v)


def _attention(Q, Qr, K, Kr, V):
    return pl.pallas_call(
        _attn_body,
        grid=(B, H // HPB),
        in_specs=[
            pl.BlockSpec((S, HPB * Dh), lambda b, h: (b, h)),
            pl.BlockSpec((S, HPB * Dr), lambda b, h: (b, h)),
            pl.BlockSpec((S, HPB * Dh), lambda b, h: (b, h)),
            pl.BlockSpec((S, Dr), lambda b, h: (b, 0)),
            pl.BlockSpec((S, HPB * Dh), lambda b, h: (b, h)),
        ],
        out_specs=pl.BlockSpec((S, HPB * Dh), lambda b, h: (b, h)),
        out_shape=jax.ShapeDtypeStruct((M, D), jnp.float32),
    )(Q, Qr, K, Kr, V)


def kernel(x, Wdkv, Wuk, Wuv, Wq, Wqr, Wkr, Wo):
    x2 = x.reshape(M, D)
    cz = _gemm(x2, Wdkv)
    c_all, wuk_f, wuv_f = _gather(cz, Wuk, Wuv)
    Q = _gemm(x2, Wq, 256)
    Qr = _gemm(x2, Wqr, 256)
    Kr = _gemm(x2, Wkr)
    K = _kv(c_all, wuk_f)
    V = _kv(c_all, wuv_f)
    O = _attention(Q, Qr, K, Kr, V)
    out = _gemm(O, Wo, 256)
    return out.reshape(B, S, D)


# device time: 353110 ns/iter; 1.1884x vs baseline; 1.1884x over previous
import jax
import jax.numpy as jnp
from jax import lax
from jax.experimental import pallas as pl
from jax.experimental.pallas import tpu as pltpu

NZ = 4
B, S, H, Dh, Dr = 4, 256, 32, 128, 64
D = 4096
DC = 512
DCZ = DC // NZ
M = B * S
SCALE = (Dh + Dr) ** -0.5


def _dot(a, b, dn=(((1,), (0,)), ((), ()))):
    return lax.dot_general(
        a, b, dn,
        precision=lax.Precision.DEFAULT,
        preferred_element_type=jnp.float32,
    )


def _gemm_body(x_ref, w_ref, o_ref):
    o_ref[...] = _dot(x_ref[...], w_ref[...])


def _gemm(x, w, block_n=None):
    m, k = x.shape
    _, n = w.shape
    if block_n is None:
        block_n = n
    return pl.pallas_call(
        _gemm_body,
        grid=(n // block_n,),
        in_specs=[
            pl.BlockSpec((m, k), lambda j: (0, 0)),
            pl.BlockSpec((k, block_n), lambda j: (0, j)),
        ],
        out_specs=pl.BlockSpec((m, block_n), lambda j: (0, j)),
        out_shape=jax.ShapeDtypeStruct((m, n), jnp.float32),
    )(x, w)


def _gather_body(cz_ref, wuk_ref, wuv_ref, c_out, wuk_out, wuv_out,
                 send_c, recv_c, send_k, recv_k, send_v, recv_v):
    mx = lax.axis_index("x")
    my = lax.axis_index("y")
    mz = lax.axis_index("z")
    right = (mz + 1) % NZ
    left = (mz + NZ - 1) % NZ

    barrier = pltpu.get_barrier_semaphore()
    for nbr in (left, right):
        pl.semaphore_signal(
            barrier, inc=1,
            device_id=(mx, my, nbr),
            device_id_type=pl.DeviceIdType.MESH,
        )
    pl.semaphore_wait(barrier, 2)

    c_out[mz] = cz_ref[...].astype(jnp.bfloat16)
    wuk_out[pl.ds(mz * DCZ, DCZ), :] = wuk_ref[...].astype(jnp.bfloat16)
    wuv_out[pl.ds(mz * DCZ, DCZ), :] = wuv_ref[...].astype(jnp.bfloat16)

    for h in range(NZ - 1):
        src = (mz + NZ - h) % NZ
        rd_c = pltpu.make_async_remote_copy(
            src_ref=c_out.at[src],
            dst_ref=c_out.at[src],
            send_sem=send_c.at[h],
            recv_sem=recv_c.at[h],
            device_id=(mx, my, right),
            device_id_type=pl.DeviceIdType.MESH,
        )
        rd_k = pltpu.make_async_remote_copy(
            src_ref=wuk_out.at[pl.ds(src * DCZ, DCZ)],
            dst_ref=wuk_out.at[pl.ds(src * DCZ, DCZ)],
            send_sem=send_k.at[h],
            recv_sem=recv_k.at[h],
            device_id=(mx, my, right),
            device_id_type=pl.DeviceIdType.MESH,
        )
        rd_v = pltpu.make_async_remote_copy(
            src_ref=wuv_out.at[pl.ds(src * DCZ, DCZ)],
            dst_ref=wuv_out.at[pl.ds(src * DCZ, DCZ)],
            send_sem=send_v.at[h],
            recv_sem=recv_v.at[h],
            device_id=(mx, my, right),
            device_id_type=pl.DeviceIdType.MESH,
        )
        rd_c.start()
        rd_k.start()
        rd_v.start()
        rd_c.wait()
        rd_k.wait()
        rd_v.wait()


def _gather(cz, wuk_z, wuv_z):
    return pl.pallas_call(
        _gather_body,
        out_shape=[
            jax.ShapeDtypeStruct((NZ, M, DCZ), jnp.bfloat16),
            jax.ShapeDtypeStruct((DC, D), jnp.bfloat16),
            jax.ShapeDtypeStruct((DC, D), jnp.bfloat16),
        ],
        in_specs=[pl.BlockSpec(memory_space=pltpu.VMEM)] * 3,
        out_specs=[pl.BlockSpec(memory_space=pltpu.VMEM)] * 3,
        scratch_shapes=[pltpu.SemaphoreType.DMA((NZ - 1,))] * 6,
        compiler_params=pltpu.CompilerParams(collective_id=0),
    )(cz, wuk_z, wuv_z)


def _kv_body(c_ref, w_ref, o_ref):
    acc = _dot(c_ref[0], w_ref[pl.ds(0, DCZ), :])
    for z in range(1, NZ):
        acc = acc + _dot(c_ref[z], w_ref[pl.ds(z * DCZ, DCZ), :])
    o_ref[...] = acc


def _kv(c_all, w, block_n=1024):
    return pl.pallas_call(
        _kv_body,
        grid=(D // block_n,),
        in_specs=[
            pl.BlockSpec((NZ, M, DCZ), lambda j: (0, 0, 0)),
            pl.BlockSpec((DC, block_n), lambda j: (0, j)),
        ],
        out_specs=pl.BlockSpec((M, block_n), lambda j: (0, j)),
        out_shape=jax.ShapeDtypeStruct((M, D), jnp.float32),
    )(c_all, w)


HPB = 2


def _attn_body(q_ref, qr_ref, k_ref, kr_ref, v_ref, o_ref):
    dn_t = (((1,), (1,)), ((), ()))
    kr = kr_ref[...]
    for i in range(HPB):
        q = q_ref[:, i * Dh:(i + 1) * Dh]
        qr = qr_ref[:, i * Dr:(i + 1) * Dr]
        k = k_ref[:, i * Dh:(i + 1) * Dh]
        v = v_ref[:, i * Dh:(i + 1) * Dh]
        s = (_dot(q, k, dn_t) + _dot(qr, kr, dn_t)) * SCALE
        m_ = jnp.max(s, axis=1, keepdims=True)
        p = jnp.exp(s - m_)
        p = p / jnp.sum(p, axis=1, keepdims=True)
        o_ref[:, i * Dh:(i + 1) * Dh] = _dot(p, v)


def _attention(Q, Qr, K, Kr, V):
    return pl.pallas_call(
        _attn_body,
        grid=(B, H // HPB),
        in_specs=[
            pl.BlockSpec((S, HPB * Dh), lambda b, h: (b, h)),
            pl.BlockSpec((S, HPB * Dr), lambda b, h: (b, h)),
            pl.BlockSpec((S, HPB * Dh), lambda b, h: (b, h)),
            pl.BlockSpec((S, Dr), lambda b, h: (b, 0)),
            pl.BlockSpec((S, HPB * Dh), lambda b, h: (b, h)),
        ],
        out_specs=pl.BlockSpec((S, HPB * Dh), lambda b, h: (b, h)),
        out_shape=jax.ShapeDtypeStruct((M, D), jnp.float32),
    )(Q, Qr, K, Kr, V)


def kernel(x, Wdkv, Wuk, Wuv, Wq, Wqr, Wkr, Wo):
    x2 = x.reshape(M, D)
    cz = _gemm(x2, Wdkv)
    c_all, wuk_f, wuv_f = _gather(cz, Wuk, Wuv)
    Q = _gemm(x2, Wq, 256)
    Qr = _gemm(x2, Wqr, 256)
    Kr = _gemm(x2, Wkr)
    K = _kv(c_all, wuk_f)
    V = _kv(c_all, wuv_f)
    O = _attention(Q, Qr, K, Kr, V)
    out = _gemm(O, Wo, 256)
    return out.reshape(B, S, D)


# device time: 338387 ns/iter; 1.2402x vs baseline; 1.0435x over previous
import jax
import jax.numpy as jnp
from jax import lax
from jax.experimental import pallas as pl
from jax.experimental.pallas import tpu as pltpu

NZ = 4
B, S, H, Dh, Dr = 4, 256, 32, 128, 64
D = 4096
DC = 512
DCZ = DC // NZ
M = B * S
SCALE = (Dh + Dr) ** -0.5


def _dot(a, b, dn=(((1,), (0,)), ((), ()))):
    return lax.dot_general(
        a, b, dn,
        precision=lax.Precision.DEFAULT,
        preferred_element_type=jnp.float32,
    )


def _gemm_body(x_ref, w_ref, o_ref):
    o_ref[...] = _dot(x_ref[...], w_ref[...])


def _gemm(x, w, block_n=None):
    m, k = x.shape
    _, n = w.shape
    if block_n is None:
        block_n = n
    return pl.pallas_call(
        _gemm_body,
        grid=(n // block_n,),
        in_specs=[
            pl.BlockSpec((m, k), lambda j: (0, 0)),
            pl.BlockSpec((k, block_n), lambda j: (0, j)),
        ],
        out_specs=pl.BlockSpec((m, block_n), lambda j: (0, j)),
        out_shape=jax.ShapeDtypeStruct((m, n), jnp.float32),
    )(x, w)


def _gather_body(cz_ref, wuk_ref, wuv_ref, c_out, wuk_out, wuv_out,
                 send_c, recv_c, send_k, recv_k, send_v, recv_v):
    mx = lax.axis_index("x")
    my = lax.axis_index("y")
    mz = lax.axis_index("z")
    right = (mz + 1) % NZ
    left = (mz + NZ - 1) % NZ

    barrier = pltpu.get_barrier_semaphore()
    for nbr in (left, right):
        pl.semaphore_signal(
            barrier, inc=1,
            device_id=(mx, my, nbr),
            device_id_type=pl.DeviceIdType.MESH,
        )
    pl.semaphore_wait(barrier, 2)

    c_out[mz] = cz_ref[...].astype(jnp.bfloat16)
    wuk_out[pl.ds(mz * DCZ, DCZ), :] = wuk_ref[...].astype(jnp.bfloat16)
    wuv_out[pl.ds(mz * DCZ, DCZ), :] = wuv_ref[...].astype(jnp.bfloat16)

    for h in range(NZ - 1):
        src = (mz + NZ - h) % NZ
        rd_c = pltpu.make_async_remote_copy(
            src_ref=c_out.at[src],
            dst_ref=c_out.at[src],
            send_sem=send_c.at[h],
            recv_sem=recv_c.at[h],
            device_id=(mx, my, right),
            device_id_type=pl.DeviceIdType.MESH,
        )
        rd_k = pltpu.make_async_remote_copy(
            src_ref=wuk_out.at[pl.ds(src * DCZ, DCZ)],
            dst_ref=wuk_out.at[pl.ds(src * DCZ, DCZ)],
            send_sem=send_k.at[h],
            recv_sem=recv_k.at[h],
            device_id=(mx, my, right),
            device_id_type=pl.DeviceIdType.MESH,
        )
        rd_v = pltpu.make_async_remote_copy(
            src_ref=wuv_out.at[pl.ds(src * DCZ, DCZ)],
            dst_ref=wuv_out.at[pl.ds(src * DCZ, DCZ)],
            send_sem=send_v.at[h],
            recv_sem=recv_v.at[h],
            device_id=(mx, my, right),
            device_id_type=pl.DeviceIdType.MESH,
        )
        rd_c.start()
        rd_k.start()
        rd_v.start()
        rd_c.wait()
        rd_k.wait()
        rd_v.wait()


def _gather(cz, wuk_z, wuv_z):
    return pl.pallas_call(
        _gather_body,
        out_shape=[
            jax.ShapeDtypeStruct((NZ, M, DCZ), jnp.bfloat16),
            jax.ShapeDtypeStruct((DC, D), jnp.bfloat16),
            jax.ShapeDtypeStruct((DC, D), jnp.bfloat16),
        ],
        in_specs=[pl.BlockSpec(memory_space=pltpu.VMEM)] * 3,
        out_specs=[pl.BlockSpec(memory_space=pltpu.VMEM)] * 3,
        scratch_shapes=[pltpu.SemaphoreType.DMA((NZ - 1,))] * 6,
        compiler_params=pltpu.CompilerParams(collective_id=0),
    )(cz, wuk_z, wuv_z)


def _kv_body(c_ref, w_ref, o_ref):
    acc = _dot(c_ref[0], w_ref[pl.ds(0, DCZ), :])
    for z in range(1, NZ):
        acc = acc + _dot(c_ref[z], w_ref[pl.ds(z * DCZ, DCZ), :])
    o_ref[...] = acc


def _kv(c_all, w, block_n=1024):
    return pl.pallas_call(
        _kv_body,
        grid=(D // block_n,),
        in_specs=[
            pl.BlockSpec((NZ, M, DCZ), lambda j: (0, 0, 0)),
            pl.BlockSpec((DC, block_n), lambda j: (0, j)),
        ],
        out_specs=pl.BlockSpec((M, block_n), lambda j: (0, j)),
        out_shape=jax.ShapeDtypeStruct((M, D), jnp.float32),
    )(c_all, w)


HPB = 8


def _attn_body(q_ref, qr_ref, k_ref, kr_ref, v_ref, o_ref):
    dn_t = (((1,), (1,)), ((), ()))
    kr = kr_ref[...]
    for i in range(HPB):
        q = q_ref[:, i * Dh:(i + 1) * Dh]
        qr = qr_ref[:, i * Dr:(i + 1) * Dr]
        k = k_ref[:, i * Dh:(i + 1) * Dh]
        v = v_ref[:, i * Dh:(i + 1) * Dh]
        s = (_dot(q, k, dn_t) + _dot(qr, kr, dn_t)) * SCALE
        m_ = jnp.max(s, axis=1, keepdims=True)
        p = jnp.exp(s - m_)
        p = p / jnp.sum(p, axis=1, keepdims=True)
        o_ref[:, i * Dh:(i + 1) * Dh] = _dot(p, v)


def _attention(Q, Qr, K, Kr, V):
    return pl.pallas_call(
        _attn_body,
        grid=(B, H // HPB),
        in_specs=[
            pl.BlockSpec((S, HPB * Dh), lambda b, h: (b, h)),
            pl.BlockSpec((S, HPB * Dr), lambda b, h: (b, h)),
            pl.BlockSpec((S, HPB * Dh), lambda b, h: (b, h)),
            pl.BlockSpec((S, Dr), lambda b, h: (b, 0)),
            pl.BlockSpec((S, HPB * Dh), lambda b, h: (b, h)),
        ],
        out_specs=pl.BlockSpec((S, HPB * Dh), lambda b, h: (b, h)),
        out_shape=jax.ShapeDtypeStruct((M, D), jnp.float32),
    )(Q, Qr, K, Kr, V)


def kernel(x, Wdkv, Wuk, Wuv, Wq, Wqr, Wkr, Wo):
    x2 = x.reshape(M, D)
    cz = _gemm(x2, Wdkv)
    c_all, wuk_f, wuv_f = _gather(cz, Wuk, Wuv)
    Q = _gemm(x2, Wq, 256)
    Qr = _gemm(x2, Wqr, 256)
    Kr = _gemm(x2, Wkr)
    K = _kv(c_all, wuk_f)
    V = _kv(c_all, wuv_f)
    O = _attention(Q, Qr, K, Kr, V)
    out = _gemm(O, Wo, 256)
    return out.reshape(B, S, D)


# device time: 314628 ns/iter; 1.3338x vs baseline; 1.0755x over previous
import functools

import jax
import jax.numpy as jnp
from jax import lax
from jax.experimental import pallas as pl
from jax.experimental.pallas import tpu as pltpu

NZ = 4
B, S, H, Dh, Dr = 4, 256, 32, 128, 64
D = 4096
DC = 512
DCZ = DC // NZ
M = B * S
SCALE = (Dh + Dr) ** -0.5


def _dot(a, b, dn=(((1,), (0,)), ((), ()))):
    return lax.dot_general(
        a, b, dn,
        precision=lax.Precision.DEFAULT,
        preferred_element_type=jnp.float32,
    )


def _gemm_body(x_ref, w_ref, o_ref, *, scale=None):
    r = _dot(x_ref[...], w_ref[...])
    o_ref[...] = r if scale is None else r * scale


def _gemm(x, w, block_n=None, scale=None):
    m, k = x.shape
    _, n = w.shape
    if block_n is None:
        block_n = n
    return pl.pallas_call(
        functools.partial(_gemm_body, scale=scale),
        grid=(n // block_n,),
        in_specs=[
            pl.BlockSpec((m, k), lambda j: (0, 0)),
            pl.BlockSpec((k, block_n), lambda j: (0, j)),
        ],
        out_specs=pl.BlockSpec((m, block_n), lambda j: (0, j)),
        out_shape=jax.ShapeDtypeStruct((m, n), jnp.float32),
    )(x, w)


def _gather_body(cz_ref, wuk_ref, wuv_ref, c_out, wuk_out, wuv_out,
                 send_c, recv_c, send_k, recv_k, send_v, recv_v):
    mx = lax.axis_index("x")
    my = lax.axis_index("y")
    mz = lax.axis_index("z")
    right = (mz + 1) % NZ
    left = (mz + NZ - 1) % NZ

    barrier = pltpu.get_barrier_semaphore()
    for nbr in (left, right):
        pl.semaphore_signal(
            barrier, inc=1,
            device_id=(mx, my, nbr),
            device_id_type=pl.DeviceIdType.MESH,
        )
    pl.semaphore_wait(barrier, 2)

    c_out[mz] = cz_ref[...].astype(jnp.bfloat16)
    wuk_out[pl.ds(mz * DCZ, DCZ), :] = wuk_ref[...].astype(jnp.bfloat16)
    wuv_out[pl.ds(mz * DCZ, DCZ), :] = wuv_ref[...].astype(jnp.bfloat16)

    for h in range(NZ - 1):
        src = (mz + NZ - h) % NZ
        rd_c = pltpu.make_async_remote_copy(
            src_ref=c_out.at[src],
            dst_ref=c_out.at[src],
            send_sem=send_c.at[h],
            recv_sem=recv_c.at[h],
            device_id=(mx, my, right),
            device_id_type=pl.DeviceIdType.MESH,
        )
        rd_k = pltpu.make_async_remote_copy(
            src_ref=wuk_out.at[pl.ds(src * DCZ, DCZ)],
            dst_ref=wuk_out.at[pl.ds(src * DCZ, DCZ)],
            send_sem=send_k.at[h],
            recv_sem=recv_k.at[h],
            device_id=(mx, my, right),
            device_id_type=pl.DeviceIdType.MESH,
        )
        rd_v = pltpu.make_async_remote_copy(
            src_ref=wuv_out.at[pl.ds(src * DCZ, DCZ)],
            dst_ref=wuv_out.at[pl.ds(src * DCZ, DCZ)],
            send_sem=send_v.at[h],
            recv_sem=recv_v.at[h],
            device_id=(mx, my, right),
            device_id_type=pl.DeviceIdType.MESH,
        )
        rd_c.start()
        rd_k.start()
        rd_v.start()
        rd_c.wait()
        rd_k.wait()
        rd_v.wait()


def _gather(cz, wuk_z, wuv_z):
    return pl.pallas_call(
        _gather_body,
        out_shape=[
            jax.ShapeDtypeStruct((NZ, M, DCZ), jnp.bfloat16),
            jax.ShapeDtypeStruct((DC, D), jnp.bfloat16),
            jax.ShapeDtypeStruct((DC, D), jnp.bfloat16),
        ],
        in_specs=[pl.BlockSpec(memory_space=pltpu.VMEM)] * 3,
        out_specs=[pl.BlockSpec(memory_space=pltpu.VMEM)] * 3,
        scratch_shapes=[pltpu.SemaphoreType.DMA((NZ - 1,))] * 6,
        compiler_params=pltpu.CompilerParams(collective_id=0),
    )(cz, wuk_z, wuv_z)


def _kv_body(c_ref, w_ref, o_ref):
    acc = _dot(c_ref[0], w_ref[pl.ds(0, DCZ), :])
    for z in range(1, NZ):
        acc = acc + _dot(c_ref[z], w_ref[pl.ds(z * DCZ, DCZ), :])
    o_ref[...] = acc


def _kv(c_all, w, block_n=1024):
    return pl.pallas_call(
        _kv_body,
        grid=(D // block_n,),
        in_specs=[
            pl.BlockSpec((NZ, M, DCZ), lambda j: (0, 0, 0)),
            pl.BlockSpec((DC, block_n), lambda j: (0, j)),
        ],
        out_specs=pl.BlockSpec((M, block_n), lambda j: (0, j)),
        out_shape=jax.ShapeDtypeStruct((M, D), jnp.float32),
    )(c_all, w)


HPB = 8


def _attn_body(q_ref, qr_ref, k_ref, kr_ref, v_ref, o_ref):
    dn_t = (((1,), (1,)), ((), ()))
    kr = kr_ref[...]
    for i in range(HPB):
        q = q_ref[:, i * Dh:(i + 1) * Dh]
        qr = qr_ref[:, i * Dr:(i + 1) * Dr]
        k = k_ref[:, i * Dh:(i + 1) * Dh]
        v = v_ref[:, i * Dh:(i + 1) * Dh]
        p = jnp.exp(_dot(q, k, dn_t) + _dot(qr, kr, dn_t))
        rs = 1.0 / jnp.sum(p, axis=1, keepdims=True)
        o_ref[:, i * Dh:(i + 1) * Dh] = _dot(p, v) * rs


def _attention(Q, Qr, K, Kr, V):
    return pl.pallas_call(
        _attn_body,
        grid=(B, H // HPB),
        in_specs=[
            pl.BlockSpec((S, HPB * Dh), lambda b, h: (b, h)),
            pl.BlockSpec((S, HPB * Dr), lambda b, h: (b, h)),
            pl.BlockSpec((S, HPB * Dh), lambda b, h: (b, h)),
            pl.BlockSpec((S, Dr), lambda b, h: (b, 0)),
            pl.BlockSpec((S, HPB * Dh), lambda b, h: (b, h)),
        ],
        out_specs=pl.BlockSpec((S, HPB * Dh), lambda b, h: (b, h)),
        out_shape=jax.ShapeDtypeStruct((M, D), jnp.float32),
    )(Q, Qr, K, Kr, V)


def kernel(x, Wdkv, Wuk, Wuv, Wq, Wqr, Wkr, Wo):
    x2 = x.reshape(M, D)
    cz = _gemm(x2, Wdkv)
    c_all, wuk_f, wuv_f = _gather(cz, Wuk, Wuv)
    Q = _gemm(x2, Wq, 256, scale=SCALE)
    Qr = _gemm(x2, Wqr, 256, scale=SCALE)
    Kr = _gemm(x2, Wkr)
    K = _kv(c_all, wuk_f)
    V = _kv(c_all, wuv_f)
    O = _attention(Q, Qr, K, Kr, V)
    out = _gemm(O, Wo, 256)
    return out.reshape(B, S, D)


# device time: 275983 ns/iter; 1.5206x vs baseline; 1.1400x over previous
import functools

import jax
import jax.numpy as jnp
from jax import lax
from jax.experimental import pallas as pl
from jax.experimental.pallas import tpu as pltpu

NZ = 4
B, S, H, Dh, Dr = 4, 256, 32, 128, 64
D = 4096
DC = 512
DCZ = DC // NZ
M = B * S
SCALE = (Dh + Dr) ** -0.5

BN = 128
NQ = D // BN
NQR = 2048 // BN
NSTEP = NQ + NQR
WAIT_STEPS = (16, 32, NSTEP - 1)


def _dot(a, b, dn=(((1,), (0,)), ((), ()))):
    return lax.dot_general(
        a, b, dn,
        precision=lax.Precision.DEFAULT,
        preferred_element_type=jnp.float32,
    )


def _gemm_body(x_ref, w_ref, o_ref, *, scale=None):
    r = _dot(x_ref[...], w_ref[...])
    if scale is not None:
        r = r * scale
    o_ref[...] = r.astype(o_ref.dtype)


def _gemm(x, w, block_n=None, scale=None, out_dtype=jnp.float32):
    m, k = x.shape
    _, n = w.shape
    if block_n is None:
        block_n = n
    return pl.pallas_call(
        functools.partial(_gemm_body, scale=scale),
        grid=(n // block_n,),
        in_specs=[
            pl.BlockSpec((m, k), lambda j: (0, 0)),
            pl.BlockSpec((k, block_n), lambda j: (0, j)),
        ],
        out_specs=pl.BlockSpec((m, block_n), lambda j: (0, j)),
        out_shape=jax.ShapeDtypeStruct((m, n), out_dtype),
    )(x, w)


def _cz_body(x_ref, w_ref, cz_ref, xbf_ref):
    cz_ref[...] = _dot(x_ref[...], w_ref[...]).astype(jnp.bfloat16)
    xbf_ref[...] = (x_ref[...] * SCALE).astype(jnp.bfloat16)


def _cz(x2, wdkv):
    return pl.pallas_call(
        _cz_body,
        in_specs=[pl.BlockSpec(memory_space=pltpu.VMEM)] * 2,
        out_specs=[pl.BlockSpec(memory_space=pltpu.VMEM)] * 2,
        out_shape=[
            jax.ShapeDtypeStruct((M, DCZ), jnp.bfloat16),
            jax.ShapeDtypeStruct((M, D), jnp.bfloat16),
        ],
    )(x2, wdkv)


def _gqr_body(xbf_ref, wq_ref, wqr_ref, czbf_ref, wukz_ref, wuvz_ref,
              qqr_ref, c_all, wukf, wuvf,
              send_c, recv_c, send_k, recv_k, send_v, recv_v):
    j = pl.program_id(0)
    mx = lax.axis_index("x")
    my = lax.axis_index("y")
    mz = lax.axis_index("z")
    right = (mz + 1) % NZ
    left = (mz + NZ - 1) % NZ

    def hop_rdmas(h):
        src = (mz + NZ - h) % NZ
        mk = functools.partial(
            pltpu.make_async_remote_copy,
            device_id=(mx, my, right),
            device_id_type=pl.DeviceIdType.MESH,
        )
        return (
            mk(src_ref=c_all.at[src], dst_ref=c_all.at[src],
               send_sem=send_c.at[h], recv_sem=recv_c.at[h]),
            mk(src_ref=wukf.at[pl.ds(src * DCZ, DCZ)],
               dst_ref=wukf.at[pl.ds(src * DCZ, DCZ)],
               send_sem=send_k.at[h], recv_sem=recv_k.at[h]),
            mk(src_ref=wuvf.at[pl.ds(src * DCZ, DCZ)],
               dst_ref=wuvf.at[pl.ds(src * DCZ, DCZ)],
               send_sem=send_v.at[h], recv_sem=recv_v.at[h]),
        )

    @pl.when(j == 0)
    def _():
        barrier = pltpu.get_barrier_semaphore()
        for nbr in (left, right):
            pl.semaphore_signal(
                barrier, inc=1,
                device_id=(mx, my, nbr),
                device_id_type=pl.DeviceIdType.MESH,
            )
        pl.semaphore_wait(barrier, 2)

        c_all[mz] = czbf_ref[...]
        wukf[pl.ds(mz * DCZ, DCZ), :] = wukz_ref[...].astype(jnp.bfloat16)
        wuvf[pl.ds(mz * DCZ, DCZ), :] = wuvz_ref[...].astype(jnp.bfloat16)
        for r in hop_rdmas(0):
            r.start()

    for h in range(1, NZ):
        @pl.when(j == WAIT_STEPS[h - 1])
        def _(h=h):
            for r in hop_rdmas(h - 1):
                r.wait()
            if h < NZ - 1:
                for r in hop_rdmas(h):
                    r.start()

    @pl.when(j < NQ)
    def _():
        qqr_ref[...] = _dot(
            xbf_ref[...], wq_ref[...].astype(jnp.bfloat16)
        ).astype(jnp.bfloat16)

    @pl.when(j >= NQ)
    def _():
        qqr_ref[...] = _dot(
            xbf_ref[...], wqr_ref[...].astype(jnp.bfloat16)
        ).astype(jnp.bfloat16)


def _gather_qqr(xbf, wq, wqr, czbf, wuk_z, wuv_z):
    vmem = pl.BlockSpec(memory_space=pltpu.VMEM)
    return pl.pallas_call(
        _gqr_body,
        grid=(NSTEP,),
        in_specs=[
            vmem,
            pl.BlockSpec((D, BN), lambda j: (0, jnp.minimum(j, NQ - 1))),
            pl.BlockSpec((D, BN), lambda j: (0, jnp.maximum(j - NQ, 0))),
            vmem,
            vmem,
            vmem,
        ],
        out_specs=[
            pl.BlockSpec((M, BN), lambda j: (0, j)),
            vmem, vmem, vmem,
        ],
        out_shape=[
            jax.ShapeDtypeStruct((M, D + 2048), jnp.bfloat16),
            jax.ShapeDtypeStruct((NZ, M, DCZ), jnp.bfloat16),
            jax.ShapeDtypeStruct((DC, D), jnp.bfloat16),
            jax.ShapeDtypeStruct((DC, D), jnp.bfloat16),
        ],
        scratch_shapes=[pltpu.SemaphoreType.DMA((NZ - 1,))] * 6,
        compiler_params=pltpu.CompilerParams(
            collective_id=0,
            dimension_semantics=("arbitrary",),
        ),
    )(xbf, wq, wqr, czbf, wuk_z, wuv_z)


def _kv_body(c_ref, w_ref, o_ref):
    acc = _dot(c_ref[0], w_ref[pl.ds(0, DCZ), :])
    for z in range(1, NZ):
        acc = acc + _dot(c_ref[z], w_ref[pl.ds(z * DCZ, DCZ), :])
    o_ref[...] = acc.astype(o_ref.dtype)


def _kv(c_all, w, block_n=1024, out_dtype=jnp.float32):
    return pl.pallas_call(
        _kv_body,
        grid=(D // block_n,),
        in_specs=[
            pl.BlockSpec((NZ, M, DCZ), lambda j: (0, 0, 0)),
            pl.BlockSpec((DC, block_n), lambda j: (0, j)),
        ],
        out_specs=pl.BlockSpec((M, block_n), lambda j: (0, j)),
        out_shape=jax.ShapeDtypeStruct((M, D), out_dtype),
    )(c_all, w)


HPB = 8


def _attn_body(q_ref, qr_ref, k_ref, kr_ref, v_ref, o_ref):
    dn_t = (((1,), (1,)), ((), ()))
    kr = kr_ref[...]
    for i in range(HPB):
        q = q_ref[:, i * Dh:(i + 1) * Dh]
        qr = qr_ref[:, i * Dr:(i + 1) * Dr]
        k = k_ref[:, i * Dh:(i + 1) * Dh]
        v = v_ref[:, i * Dh:(i + 1) * Dh]
        p = jnp.exp(_dot(q, k, dn_t) + _dot(qr, kr, dn_t))
        rs = 1.0 / jnp.sum(p, axis=1, keepdims=True)
        o_ref[:, i * Dh:(i + 1) * Dh] = _dot(p, v) * rs


def _attention(QQr, K, Kr, V):
    qr_off = D // (HPB * Dr)
    return pl.pallas_call(
        _attn_body,
        grid=(B, H // HPB),
        in_specs=[
            pl.BlockSpec((S, HPB * Dh), lambda b, h: (b, h)),
            pl.BlockSpec((S, HPB * Dr), lambda b, h: (b, h + qr_off)),
            pl.BlockSpec((S, HPB * Dh), lambda b, h: (b, h)),
            pl.BlockSpec((S, Dr), lambda b, h: (b, 0)),
            pl.BlockSpec((S, HPB * Dh), lambda b, h: (b, h)),
        ],
        out_specs=pl.BlockSpec((S, HPB * Dh), lambda b, h: (b, h)),
        out_shape=jax.ShapeDtypeStruct((M, D), jnp.float32),
    )(QQr, QQr, K, Kr, V)


def kernel(x, Wdkv, Wuk, Wuv, Wq, Wqr, Wkr, Wo):
    x2 = x.reshape(M, D)
    czbf, xbf = _cz(x2, Wdkv)
    QQr, c_all, wuk_f, wuv_f = _gather_qqr(xbf, Wq, Wqr, czbf, Wuk, Wuv)
    Kr = _gemm(x2, Wkr, out_dtype=jnp.bfloat16)
    K = _kv(c_all, wuk_f, out_dtype=jnp.bfloat16)
    V = _kv(c_all, wuv_f)
    O = _attention(QQr, K, Kr, V)
    out = _gemm(O, Wo, 256)
    return out.reshape(B, S, D)
